# Initial kernel scaffold; baseline (speedup 1.0000x reference)
import jax
import jax.numpy as jnp
from jax.experimental import pallas as pl


def kernel(query, key_value, edge_index, node_degrees, clustering_coeffs, pos_embedding, Wc, bc, Wd, bd, Wq, bq, Wk, bk, Wv, bv, Wo, bo, ln_q_g, ln_q_b, ln_kv_g, ln_kv_b, ln_out_g, ln_out_b):
    raise NotImplementedError("write your pallas kernel here")


# speedup vs baseline: 15.9583x; 15.9583x over previous
import functools

import numpy as np
import jax
import jax.numpy as jnp
from jax import lax
from jax.experimental import pallas as pl
from jax.experimental.pallas import tpu as pltpu
from jax.experimental.pallas import tpu_sc as plsc

_N = 10000
_D = 128
_NH = 8
_HD = 16
_NB = 50
_SCALE = float(1.0 / np.sqrt(_HD))
_NW = 32
_PW = 320
_NP = _NW * _PW
_NBP = 56
_BS = 512
_EPS = 1e-5


def _ln_blk(x, g, b):
    m = jnp.mean(x, axis=-1, keepdims=True)
    v = jnp.mean((x - m) ** 2, axis=-1, keepdims=True)
    return (x - m) * jax.lax.rsqrt(v + _EPS) * g + b


def _dense_pre_body(q_ref, kv_ref, deg_ref, clus_ref, pe_ref,
                    wc_ref, bc_ref, wd_ref, bd_ref,
                    wq_ref, bq_ref, wk_ref, bk_ref, wv_ref, bv_ref,
                    g1_ref, b1_ref, g2_ref, b2_ref,
                    qpe_ref, qo_ref, ko_ref, vo_ref):
    pe = pe_ref[...]
    ce = clus_ref[...] * wc_ref[...] + bc_ref[...]
    de = deg_ref[...] * wd_ref[...] + bd_ref[...]
    penc = jnp.concatenate([pe[:, : _D // 2], ce, de], axis=-1)
    qpe = q_ref[...] + penc
    kvpe = kv_ref[...] + penc
    qn = _ln_blk(qpe, g1_ref[...], b1_ref[...])
    kvn = _ln_blk(kvpe, g2_ref[...], b2_ref[...])
    dn = (((1,), (1,)), ((), ()))
    qo_ref[...] = (lax.dot_general(qn, wq_ref[...], dn,
                                   preferred_element_type=jnp.float32)
                   + bq_ref[...]) * _SCALE
    ko_ref[...] = lax.dot_general(kvn, wk_ref[...], dn,
                                  preferred_element_type=jnp.float32) + bk_ref[...]
    vo_ref[...] = lax.dot_general(kvn, wv_ref[...], dn,
                                  preferred_element_type=jnp.float32) + bv_ref[...]
    qpe_ref[...] = qpe


def _dense_pre(qp, kvp, degc, clusc, pep, wct, bct, wdt, bdt,
               wq, bq2, wk, bk2, wv, bv2, g1, b1, g2, b2, interpret=False):
    nblk = _NP // _BS
    row_spec = pl.BlockSpec((_BS, _D), lambda i: (i, 0))
    col_spec = pl.BlockSpec((_BS, 1), lambda i: (i, 0))

    def full(shape):
        return pl.BlockSpec(shape, lambda i: (0,) * len(shape))

    return pl.pallas_call(
        _dense_pre_body,
        grid=(nblk,),
        in_specs=[row_spec, row_spec, col_spec, col_spec, row_spec,
                  full((1, _D // 4)), full((1, _D // 4)),
                  full((1, _D // 4)), full((1, _D // 4)),
                  full((_D, _D)), full((1, _D)),
                  full((_D, _D)), full((1, _D)),
                  full((_D, _D)), full((1, _D)),
                  full((1, _D)), full((1, _D)), full((1, _D)), full((1, _D))],
        out_specs=[row_spec, row_spec, row_spec, row_spec],
        out_shape=[jax.ShapeDtypeStruct((_NP, _D), jnp.float32)] * 4,
        interpret=interpret,
    )(qp, kvp, degc, clusc, pep, wct, bct, wdt, bdt,
      wq, bq2, wk, bk2, wv, bv2, g1, b1, g2, b2)


def _dense_post_body(attn_ref, qpe_ref, wo_ref, bo_ref, g_ref, b_ref, o_ref):
    dn = (((1,), (1,)), ((), ()))
    out = lax.dot_general(attn_ref[...], wo_ref[...], dn,
                          preferred_element_type=jnp.float32) + bo_ref[...]
    o_ref[...] = _ln_blk(qpe_ref[...] + out, g_ref[...], b_ref[...])


def _dense_post(attn, qpe, wo, bo2, g, b, interpret=False):
    nblk = _NP // _BS
    row_spec = pl.BlockSpec((_BS, _D), lambda i: (i, 0))

    def full(shape):
        return pl.BlockSpec(shape, lambda i: (0,) * len(shape))

    return pl.pallas_call(
        _dense_post_body,
        grid=(nblk,),
        in_specs=[row_spec, row_spec, full((_D, _D)), full((1, _D)),
                  full((1, _D)), full((1, _D))],
        out_specs=row_spec,
        out_shape=jax.ShapeDtypeStruct((_NP, _D), jnp.float32),
        interpret=interpret,
    )(attn, qpe, wo, bo2, g, b)


def _build_table(edge_index):
    r, c = edge_index[0], edge_index[1]
    loops = jnp.arange(_N, dtype=r.dtype)
    row = jnp.concatenate([r, c, loops])
    col = jnp.concatenate([c, r, loops])
    order = jnp.argsort(row)
    row_s = jnp.take(row, order).astype(jnp.int32)
    col_s = jnp.take(col, order).astype(jnp.int32)
    deg = jnp.zeros((_NP,), jnp.int32).at[row_s].add(1)
    starts = jnp.concatenate([jnp.zeros((1,), jnp.int32),
                              jnp.cumsum(deg)])[:_NP]
    rank = (jnp.arange(row_s.shape[0], dtype=jnp.int32)
            - jnp.take(starts, row_s))
    base = jnp.broadcast_to(jnp.arange(_NP, dtype=jnp.int32)[:, None],
                            (_NP, _NBP))
    tbl = base.at[row_s, rank].add(col_s - row_s, mode="drop",
                                   unique_indices=True)
    cnt = jnp.minimum(deg, _NB)
    return tbl, cnt


def _sc_attention(q, k, v, tbl, cnt):
    mesh = plsc.VectorSubcoreMesh(core_axis_name="c", subcore_axis_name="s")

    @functools.partial(
        pl.kernel,
        mesh=mesh,
        out_type=jax.ShapeDtypeStruct((_NP, _D), jnp.float32),
        compiler_params=pltpu.CompilerParams(needs_layout_passes=False),
        scratch_types=[
            pltpu.VMEM((_PW + 16,), jnp.int32),
            pltpu.VMEM((_NBP,), jnp.int32),
            pltpu.VMEM((_NBP, _D), jnp.float32),
            pltpu.VMEM((_NBP, _D), jnp.float32),
            pltpu.VMEM((_D,), jnp.float32),
            pltpu.VMEM((_NH * 64 + 16,), jnp.float32),
            pltpu.VMEM((_D,), jnp.float32),
            pltpu.SemaphoreType.DMA,
            pltpu.SemaphoreType.DMA,
        ],
    )
    def body(q_hbm, k_hbm, v_hbm, tbl_hbm, cnt_hbm, out_hbm,
             cnt_v, idx_v, kbuf, vbuf, qbuf, sbuf, obuf, semk, semv):
        wid = lax.axis_index("s") * 2 + lax.axis_index("c")
        base = wid * _PW
        pltpu.sync_copy(cnt_hbm.at[pl.ds(base, _PW)],
                        cnt_v.at[pl.ds(0, _PW)])
        lane0 = lax.iota(jnp.int32, 16) == 0

        def node_body(i, carry):
            u = base + i
            pltpu.sync_copy(tbl_hbm.at[u], idx_v)
            pltpu.sync_copy(q_hbm.at[u], qbuf)
            ck = pltpu.async_copy(k_hbm.at[idx_v], kbuf, semk)
            cv = pltpu.async_copy(v_hbm.at[idx_v], vbuf, semv)
            cntv = jnp.full((16,), cnt_v[pl.ds(i, 16)][0], dtype=jnp.int32)
            qh = [qbuf[pl.ds(h * _HD, _HD)] for h in range(_NH)]
            ck.wait()

            def score_body(j, c2):
                for h in range(_NH):
                    kv = kbuf[j, pl.ds(h * _HD, _HD)]
                    s = jnp.sum(kv * qh[h])
                    plsc.store_scatter(sbuf,
                                       [jnp.full((16,), h * 64 + j,
                                                 dtype=jnp.int32)],
                                       jnp.full((16,), s), mask=lane0)
                return c2
            lax.fori_loop(0, _NB, score_body, 0)

            cv.wait()
            for h in range(_NH):
                svs = []
                for e in range(4):
                    lane = lax.iota(jnp.int32, 16) + (16 * e)
                    sv = sbuf[pl.ds(h * 64 + 16 * e, 16)]
                    svs.append(jnp.where(lane < cntv, sv,
                                         jnp.full((16,), -1e30)))
                m = jnp.max(jnp.maximum(jnp.maximum(svs[0], svs[1]),
                                        jnp.maximum(svs[2], svs[3])))
                exs = [jnp.exp(sv - m) for sv in svs]
                tot = jnp.sum(exs[0] + exs[1] + exs[2] + exs[3])
                for e in range(4):
                    sbuf[pl.ds(h * 64 + 16 * e, 16)] = exs[e] / tot

            def v_body(j, accs):
                out = []
                for h in range(_NH):
                    a = sbuf[pl.ds(h * 64 + j, 16)][0]
                    vv = vbuf[j, pl.ds(h * _HD, _HD)]
                    out.append(accs[h] + vv * a)
                return tuple(out)
            accs = lax.fori_loop(
                0, _NB, v_body,
                tuple(jnp.zeros((16,), jnp.float32) for _ in range(_NH)))
            for h in range(_NH):
                obuf[pl.ds(h * _HD, _HD)] = accs[h]
            pltpu.sync_copy(obuf, out_hbm.at[u])
            return carry

        lax.fori_loop(0, _PW, node_body, 0)

    return body(q, k, v, tbl, cnt)


def kernel(query, key_value, edge_index, node_degrees, clustering_coeffs,
           pos_embedding, Wc, bc, Wd, bd, Wq, bq, Wk, bk, Wv, bv, Wo, bo,
           ln_q_g, ln_q_b, ln_kv_g, ln_kv_b, ln_out_g, ln_out_b):
    pad = _NP - _N
    qp = jnp.pad(query, ((0, pad), (0, 0)))
    kvp = jnp.pad(key_value, ((0, pad), (0, 0)))
    pep = jnp.pad(pos_embedding, ((0, pad), (0, 0)))
    degc = jnp.pad(node_degrees, (0, pad))[:, None]
    clusc = jnp.pad(clustering_coeffs, (0, pad))[:, None]

    tbl, cnt = _build_table(edge_index)

    qpe, q, k, v = _dense_pre(
        qp, kvp, degc, clusc, pep,
        Wc.reshape(1, _D // 4), bc.reshape(1, _D // 4),
        Wd.reshape(1, _D // 4), bd.reshape(1, _D // 4),
        Wq, bq.reshape(1, _D), Wk, bk.reshape(1, _D),
        Wv, bv.reshape(1, _D),
        ln_q_g.reshape(1, _D), ln_q_b.reshape(1, _D),
        ln_kv_g.reshape(1, _D), ln_kv_b.reshape(1, _D))

    attn = _sc_attention(q, k, v, tbl, cnt)

    out = _dense_post(attn, qpe, Wo, bo.reshape(1, _D),
                      ln_out_g.reshape(1, _D), ln_out_b.reshape(1, _D))
    return out[:_N]


# speedup vs baseline: 18.8519x; 1.1813x over previous
import functools

import numpy as np
import jax
import jax.numpy as jnp
from jax import lax
from jax.experimental import pallas as pl
from jax.experimental.pallas import tpu as pltpu
from jax.experimental.pallas import tpu_sc as plsc

_N = 10000
_D = 128
_NH = 8
_HD = 16
_NB = 50
_SCALE = float(1.0 / np.sqrt(_HD))
_NW = 32
_PW = 320
_NP = _NW * _PW
_NBP = 56
_BS = 512
_EPS = 1e-5


def _ln_blk(x, g, b):
    m = jnp.mean(x, axis=-1, keepdims=True)
    v = jnp.mean((x - m) ** 2, axis=-1, keepdims=True)
    return (x - m) * jax.lax.rsqrt(v + _EPS) * g + b


def _dense_pre_body(q_ref, kv_ref, deg_ref, clus_ref, pe_ref,
                    wc_ref, bc_ref, wd_ref, bd_ref,
                    wq_ref, bq_ref, wk_ref, bk_ref, wv_ref, bv_ref,
                    g1_ref, b1_ref, g2_ref, b2_ref,
                    qpe_ref, qo_ref, ko_ref, vo_ref):
    pe = pe_ref[...]
    ce = clus_ref[...] * wc_ref[...] + bc_ref[...]
    de = deg_ref[...] * wd_ref[...] + bd_ref[...]
    penc = jnp.concatenate([pe[:, : _D // 2], ce, de], axis=-1)
    qpe = q_ref[...] + penc
    kvpe = kv_ref[...] + penc
    qn = _ln_blk(qpe, g1_ref[...], b1_ref[...])
    kvn = _ln_blk(kvpe, g2_ref[...], b2_ref[...])
    dn = (((1,), (1,)), ((), ()))
    qo_ref[...] = (lax.dot_general(qn, wq_ref[...], dn,
                                   preferred_element_type=jnp.float32)
                   + bq_ref[...]) * _SCALE
    ko_ref[...] = lax.dot_general(kvn, wk_ref[...], dn,
                                  preferred_element_type=jnp.float32) + bk_ref[...]
    vo_ref[...] = lax.dot_general(kvn, wv_ref[...], dn,
                                  preferred_element_type=jnp.float32) + bv_ref[...]
    qpe_ref[...] = qpe


def _dense_pre(qp, kvp, degc, clusc, pep, wct, bct, wdt, bdt,
               wq, bq2, wk, bk2, wv, bv2, g1, b1, g2, b2, interpret=False):
    nblk = _NP // _BS
    row_spec = pl.BlockSpec((_BS, _D), lambda i: (i, 0))
    col_spec = pl.BlockSpec((_BS, 1), lambda i: (i, 0))

    def full(shape):
        return pl.BlockSpec(shape, lambda i: (0,) * len(shape))

    return pl.pallas_call(
        _dense_pre_body,
        grid=(nblk,),
        in_specs=[row_spec, row_spec, col_spec, col_spec, row_spec,
                  full((1, _D // 4)), full((1, _D // 4)),
                  full((1, _D // 4)), full((1, _D // 4)),
                  full((_D, _D)), full((1, _D)),
                  full((_D, _D)), full((1, _D)),
                  full((_D, _D)), full((1, _D)),
                  full((1, _D)), full((1, _D)), full((1, _D)), full((1, _D))],
        out_specs=[row_spec, row_spec, row_spec, row_spec],
        out_shape=[jax.ShapeDtypeStruct((_NP, _D), jnp.float32)] * 4,
        interpret=interpret,
    )(qp, kvp, degc, clusc, pep, wct, bct, wdt, bdt,
      wq, bq2, wk, bk2, wv, bv2, g1, b1, g2, b2)


def _dense_post_body(attn_ref, qpe_ref, wo_ref, bo_ref, g_ref, b_ref, o_ref):
    dn = (((1,), (1,)), ((), ()))
    out = lax.dot_general(attn_ref[...], wo_ref[...], dn,
                          preferred_element_type=jnp.float32) + bo_ref[...]
    o_ref[...] = _ln_blk(qpe_ref[...] + out, g_ref[...], b_ref[...])


def _dense_post(attn, qpe, wo, bo2, g, b, interpret=False):
    nblk = _NP // _BS
    row_spec = pl.BlockSpec((_BS, _D), lambda i: (i, 0))

    def full(shape):
        return pl.BlockSpec(shape, lambda i: (0,) * len(shape))

    return pl.pallas_call(
        _dense_post_body,
        grid=(nblk,),
        in_specs=[row_spec, row_spec, full((_D, _D)), full((1, _D)),
                  full((1, _D)), full((1, _D))],
        out_specs=row_spec,
        out_shape=jax.ShapeDtypeStruct((_NP, _D), jnp.float32),
        interpret=interpret,
    )(attn, qpe, wo, bo2, g, b)


def _build_table(edge_index):
    r, c = edge_index[0], edge_index[1]
    loops = jnp.arange(_N, dtype=r.dtype)
    row = jnp.concatenate([r, c, loops])
    col = jnp.concatenate([c, r, loops])
    order = jnp.argsort(row)
    row_s = jnp.take(row, order).astype(jnp.int32)
    col_s = jnp.take(col, order).astype(jnp.int32)
    deg = jnp.zeros((_NP,), jnp.int32).at[row_s].add(1)
    starts = jnp.concatenate([jnp.zeros((1,), jnp.int32),
                              jnp.cumsum(deg)])[:_NP]
    rank = (jnp.arange(row_s.shape[0], dtype=jnp.int32)
            - jnp.take(starts, row_s))
    base = jnp.broadcast_to(jnp.arange(_NP, dtype=jnp.int32)[:, None],
                            (_NP, _NBP))
    tbl = base.at[row_s, rank].add(col_s - row_s, mode="drop",
                                   unique_indices=True)
    cnt = jnp.minimum(deg, _NB)
    return tbl, cnt


def _sc_attention(q, k, v, tbl, cnt):
    mesh = plsc.VectorSubcoreMesh(core_axis_name="c", subcore_axis_name="s")

    @functools.partial(
        pl.kernel,
        mesh=mesh,
        out_type=jax.ShapeDtypeStruct((_NP, _D), jnp.float32),
        compiler_params=pltpu.CompilerParams(needs_layout_passes=False),
        scratch_types=[
            pltpu.VMEM((_PW + 16,), jnp.int32),
            pltpu.VMEM((2, _NBP), jnp.int32),
            pltpu.VMEM((2, _NBP, _D), jnp.float32),
            pltpu.VMEM((2, _NBP, _D), jnp.float32),
            pltpu.VMEM((2, _D), jnp.float32),
            pltpu.VMEM((_NH * 64 + 16,), jnp.float32),
            pltpu.VMEM((2, _D), jnp.float32),
            pltpu.SemaphoreType.DMA,
            pltpu.SemaphoreType.DMA,
            pltpu.SemaphoreType.DMA,
            pltpu.SemaphoreType.DMA,
            pltpu.SemaphoreType.DMA,
            pltpu.SemaphoreType.DMA,
            pltpu.SemaphoreType.DMA,
            pltpu.SemaphoreType.DMA,
            pltpu.SemaphoreType.DMA,
            pltpu.SemaphoreType.DMA,
        ],
    )
    def body(q_hbm, k_hbm, v_hbm, tbl_hbm, cnt_hbm, out_hbm,
             cnt_v, idx2, kbuf, vbuf, qbuf, sbuf, obuf,
             semt0, semt1, semq0, semq1, semk0, semk1,
             semv0, semv1, semo0, semo1):
        semt = (semt0, semt1)
        semq = (semq0, semq1)
        semk = (semk0, semk1)
        semv = (semv0, semv1)
        semo = (semo0, semo1)
        wid = lax.axis_index("s") * 2 + lax.axis_index("c")
        base = wid * _PW
        pltpu.sync_copy(cnt_hbm.at[pl.ds(base, _PW)],
                        cnt_v.at[pl.ds(0, _PW)])
        iota = lax.iota(jnp.int32, 16)

        def issue_a(n, b):
            u = base + n
            pltpu.async_copy(tbl_hbm.at[u], idx2.at[b], semt[b])
            pltpu.async_copy(q_hbm.at[u], qbuf.at[b], semq[b])

        def wait_a(b):
            pltpu.make_async_copy(tbl_hbm.at[0], idx2.at[b],
                                  semt[b]).wait()
            pltpu.make_async_copy(q_hbm.at[0], qbuf.at[b], semq[b]).wait()

        def issue_kv(b):
            pltpu.async_copy(k_hbm.at[idx2.at[b]], kbuf.at[b], semk[b])
            pltpu.async_copy(v_hbm.at[idx2.at[b]], vbuf.at[b], semv[b])

        def wait_kv(b):
            pltpu.make_async_copy(k_hbm.at[pl.ds(0, _NBP)], kbuf.at[b],
                                  semk[b]).wait()
            pltpu.make_async_copy(v_hbm.at[pl.ds(0, _NBP)], vbuf.at[b],
                                  semv[b]).wait()

        def wait_o(b):
            pltpu.make_async_copy(obuf.at[b], out_hbm.at[0],
                                  semo[b]).wait()

        issue_a(0, 0)
        wait_a(0)
        issue_kv(0)
        issue_a(1, 1)

        def one_node(i, b):
            u = base + i
            nb = 1 - b

            @pl.when(i + 1 < _PW)
            def _():
                wait_a(nb)
                issue_kv(nb)

            wait_kv(b)
            cntv = jnp.full((16,), cnt_v[pl.ds(i, 16)][0], dtype=jnp.int32)

            def score_body(he, c2):
                h = he // 4
                e = he % 4
                qv = qbuf[b, pl.ds(h * _HD, _HD)]
                rowv = iota + e * 16
                rowc = jnp.minimum(rowv, _NBP - 1)
                accs = [jnp.zeros((16,), jnp.float32) for _ in range(4)]
                for d in range(_HD):
                    col = jnp.full((16,), h * _HD + d, dtype=jnp.int32)
                    kcol = plsc.load_gather(kbuf.at[b], [rowc, col])
                    accs[d % 4] = accs[d % 4] + kcol * qv[d]
                sc = (accs[0] + accs[1]) + (accs[2] + accs[3])
                sv = jnp.where(rowv < cntv, sc, jnp.full((16,), -1e30))
                sbuf[pl.ds(he * 16, 16)] = sv
                return c2
            lax.fori_loop(0, _NH * 4, score_body, 0)

            for h in range(_NH):
                svs = [sbuf[pl.ds(h * 64 + 16 * e, 16)] for e in range(4)]
                m = jnp.max(jnp.maximum(jnp.maximum(svs[0], svs[1]),
                                        jnp.maximum(svs[2], svs[3])))
                exs = [jnp.exp(sv - m) for sv in svs]
                tot = jnp.sum(exs[0] + exs[1] + exs[2] + exs[3])
                for e in range(4):
                    sbuf[pl.ds(h * 64 + 16 * e, 16)] = exs[e] / tot

            aidx_base = (iota & 7) * 64

            def v_body(j, accs):
                av = plsc.load_gather(sbuf, [aidx_base + j])
                out = []
                for h in range(_NH):
                    vv = vbuf[b, j, pl.ds(h * _HD, _HD)]
                    out.append(accs[h] + vv * av[h])
                return tuple(out)
            accs = lax.fori_loop(
                0, _NB, v_body,
                tuple(jnp.zeros((16,), jnp.float32) for _ in range(_NH)))

            @pl.when(i >= 2)
            def _():
                wait_o(b)

            for h in range(_NH):
                obuf[b, pl.ds(h * _HD, _HD)] = accs[h]
            pltpu.async_copy(obuf.at[b], out_hbm.at[u], semo[b])

            @pl.when(i + 2 < _PW)
            def _():
                issue_a(i + 2, b)

        def pair_body(p, carry):
            one_node(p * 2, 0)
            one_node(p * 2 + 1, 1)
            return carry

        lax.fori_loop(0, _PW // 2, pair_body, 0)
        wait_o(0)
        wait_o(1)

    return body(q, k, v, tbl, cnt)


def kernel(query, key_value, edge_index, node_degrees, clustering_coeffs,
           pos_embedding, Wc, bc, Wd, bd, Wq, bq, Wk, bk, Wv, bv, Wo, bo,
           ln_q_g, ln_q_b, ln_kv_g, ln_kv_b, ln_out_g, ln_out_b):
    pad = _NP - _N
    qp = jnp.pad(query, ((0, pad), (0, 0)))
    kvp = jnp.pad(key_value, ((0, pad), (0, 0)))
    pep = jnp.pad(pos_embedding, ((0, pad), (0, 0)))
    degc = jnp.pad(node_degrees, (0, pad))[:, None]
    clusc = jnp.pad(clustering_coeffs, (0, pad))[:, None]

    tbl, cnt = _build_table(edge_index)

    qpe, q, k, v = _dense_pre(
        qp, kvp, degc, clusc, pep,
        Wc.reshape(1, _D // 4), bc.reshape(1, _D // 4),
        Wd.reshape(1, _D // 4), bd.reshape(1, _D // 4),
        Wq, bq.reshape(1, _D), Wk, bk.reshape(1, _D),
        Wv, bv.reshape(1, _D),
        ln_q_g.reshape(1, _D), ln_q_b.reshape(1, _D),
        ln_kv_g.reshape(1, _D), ln_kv_b.reshape(1, _D))

    attn = _sc_attention(q, k, v, tbl, cnt)

    out = _dense_post(attn, qpe, Wo, bo.reshape(1, _D),
                      ln_out_g.reshape(1, _D), ln_out_b.reshape(1, _D))
    return out[:_N]


# speedup vs baseline: 56.7038x; 3.0079x over previous
import functools

import numpy as np
import jax
import jax.numpy as jnp
from jax import lax
from jax.experimental import pallas as pl
from jax.experimental.pallas import tpu as pltpu
from jax.experimental.pallas import tpu_sc as plsc

_N = 10000
_D = 128
_NH = 8
_HD = 16
_NB = 50
_SCALE = float(1.0 / np.sqrt(_HD))
_NW = 32
_PW = 320
_NP = _NW * _PW
_NBP = 56
_BS = 512
_EPS = 1e-5


def _ln_blk(x, g, b):
    m = jnp.mean(x, axis=-1, keepdims=True)
    v = jnp.mean((x - m) ** 2, axis=-1, keepdims=True)
    return (x - m) * jax.lax.rsqrt(v + _EPS) * g + b


def _dense_pre_body(q_ref, kv_ref, deg_ref, clus_ref, pe_ref,
                    wc_ref, bc_ref, wd_ref, bd_ref,
                    wq_ref, bq_ref, wk_ref, bk_ref, wv_ref, bv_ref,
                    g1_ref, b1_ref, g2_ref, b2_ref,
                    qpe_ref, qo_ref, ko_ref, vo_ref):
    pe = pe_ref[...]
    ce = clus_ref[...] * wc_ref[...] + bc_ref[...]
    de = deg_ref[...] * wd_ref[...] + bd_ref[...]
    penc = jnp.concatenate([pe[:, : _D // 2], ce, de], axis=-1)
    qpe = q_ref[...] + penc
    kvpe = kv_ref[...] + penc
    qn = _ln_blk(qpe, g1_ref[...], b1_ref[...])
    kvn = _ln_blk(kvpe, g2_ref[...], b2_ref[...])
    dn = (((1,), (1,)), ((), ()))
    qo_ref[...] = (lax.dot_general(qn, wq_ref[...], dn,
                                   preferred_element_type=jnp.float32)
                   + bq_ref[...]) * _SCALE
    ko_ref[...] = lax.dot_general(kvn, wk_ref[...], dn,
                                  preferred_element_type=jnp.float32) + bk_ref[...]
    vo_ref[...] = lax.dot_general(kvn, wv_ref[...], dn,
                                  preferred_element_type=jnp.float32) + bv_ref[...]
    qpe_ref[...] = qpe


def _dense_pre(qp, kvp, degc, clusc, pep, wct, bct, wdt, bdt,
               wq, bq2, wk, bk2, wv, bv2, g1, b1, g2, b2, interpret=False):
    nblk = _NP // _BS
    row_spec = pl.BlockSpec((_BS, _D), lambda i: (i, 0))
    col_spec = pl.BlockSpec((_BS, 1), lambda i: (i, 0))

    def full(shape):
        return pl.BlockSpec(shape, lambda i: (0,) * len(shape))

    return pl.pallas_call(
        _dense_pre_body,
        grid=(nblk,),
        in_specs=[row_spec, row_spec, col_spec, col_spec, row_spec,
                  full((1, _D // 4)), full((1, _D // 4)),
                  full((1, _D // 4)), full((1, _D // 4)),
                  full((_D, _D)), full((1, _D)),
                  full((_D, _D)), full((1, _D)),
                  full((_D, _D)), full((1, _D)),
                  full((1, _D)), full((1, _D)), full((1, _D)), full((1, _D))],
        out_specs=[row_spec, row_spec, row_spec, row_spec],
        out_shape=[jax.ShapeDtypeStruct((_NP, _D), jnp.float32)] * 4,
        interpret=interpret,
    )(qp, kvp, degc, clusc, pep, wct, bct, wdt, bdt,
      wq, bq2, wk, bk2, wv, bv2, g1, b1, g2, b2)


def _dense_post_body(attn_ref, qpe_ref, wo_ref, bo_ref, g_ref, b_ref, o_ref):
    dn = (((1,), (1,)), ((), ()))
    out = lax.dot_general(attn_ref[...], wo_ref[...], dn,
                          preferred_element_type=jnp.float32) + bo_ref[...]
    o_ref[...] = _ln_blk(qpe_ref[...] + out, g_ref[...], b_ref[...])


def _dense_post(attn, qpe, wo, bo2, g, b, interpret=False):
    nblk = _NP // _BS
    row_spec = pl.BlockSpec((_BS, _D), lambda i: (i, 0))

    def full(shape):
        return pl.BlockSpec(shape, lambda i: (0,) * len(shape))

    return pl.pallas_call(
        _dense_post_body,
        grid=(nblk,),
        in_specs=[row_spec, row_spec, full((_D, _D)), full((1, _D)),
                  full((1, _D)), full((1, _D))],
        out_specs=row_spec,
        out_shape=jax.ShapeDtypeStruct((_NP, _D), jnp.float32),
        interpret=interpret,
    )(attn, qpe, wo, bo2, g, b)


_CH = 2000
_NCH = 320000 // _CH


def _sc_table(edge_index):
    mesh = plsc.VectorSubcoreMesh(core_axis_name="c", subcore_axis_name="s")

    @functools.partial(
        pl.kernel,
        mesh=mesh,
        out_type=(jax.ShapeDtypeStruct((_NP * _NBP,), jnp.int32),
                  jax.ShapeDtypeStruct((_NP,), jnp.int32)),
        compiler_params=pltpu.CompilerParams(needs_layout_passes=False),
        scratch_types=[
            pltpu.VMEM((_PW * _NBP,), jnp.int32),
            pltpu.VMEM((_PW,), jnp.int32),
            pltpu.VMEM((2 * _CH,), jnp.int32),
            pltpu.VMEM((2 * _CH,), jnp.int32),
            pltpu.SemaphoreType.DMA,
            pltpu.SemaphoreType.DMA,
            pltpu.SemaphoreType.DMA,
            pltpu.SemaphoreType.DMA,
        ],
    )
    def body(r_hbm, c_hbm, tbl_hbm, cnt_hbm, tblw, cntw, rbuf, cbuf,
             semr0, semr1, semc0, semc1):
        semr = (semr0, semr1)
        semc = (semc0, semc1)
        wid = lax.axis_index("s") * 2 + lax.axis_index("c")
        base = wid * _PW
        iota = lax.iota(jnp.int32, 16)
        basev = jnp.full((16,), base, jnp.int32)

        def init_body(n, carry):
            nv = basev + jnp.full((16,), n, jnp.int32)
            tblw[pl.ds(n * _NBP, 16)] = nv
            tblw[pl.ds(n * _NBP + 16, 16)] = nv
            tblw[pl.ds(n * _NBP + 32, 16)] = nv
            tblw[pl.ds(n * _NBP + 40, 16)] = nv
            return carry
        lax.fori_loop(0, _PW, init_body, 0)

        def czero(g, carry):
            cntw[pl.ds(g * 16, 16)] = jnp.zeros((16,), jnp.int32)
            return carry
        lax.fori_loop(0, _PW // 16, czero, 0)

        def issue(ch, b):
            pltpu.async_copy(r_hbm.at[pl.ds(ch * _CH, _CH)],
                             rbuf.at[pl.ds(b * _CH, _CH)], semr[b])
            pltpu.async_copy(c_hbm.at[pl.ds(ch * _CH, _CH)],
                             cbuf.at[pl.ds(b * _CH, _CH)], semc[b])

        def wait_b(b):
            pltpu.make_async_copy(r_hbm.at[pl.ds(0, _CH)],
                                  rbuf.at[pl.ds(b * _CH, _CH)],
                                  semr[b]).wait()
            pltpu.make_async_copy(c_hbm.at[pl.ds(0, _CH)],
                                  cbuf.at[pl.ds(b * _CH, _CH)],
                                  semc[b]).wait()

        nbv = jnp.full((16,), _NB, jnp.int32)
        pwv = jnp.full((16,), _PW, jnp.int32)
        nbpm1 = jnp.full((16,), _NBP - 1, jnp.int32)
        zv = jnp.zeros((16,), jnp.int32)

        def process_vreg(src, tgt):
            local = src - basev
            m = (local >= zv) & (local < pwv)
            lc = jnp.minimum(jnp.maximum(local, zv), pwv - 1)
            cur = plsc.load_gather(cntw, [lc])
            occ, lastm = plsc.scan_count(lc, mask=m)
            rank = cur + occ - 1
            keep = m & (rank < nbv)
            rankc = jnp.minimum(rank, nbpm1)
            plsc.store_scatter(tblw, [lc * _NBP + rankc], tgt, mask=keep)
            plsc.store_scatter(cntw, [lc], cur + occ, mask=m & lastm)

        def run_part(part):
            issue(0, 0)

            def one_chunk(ch, b):
                @pl.when(ch + 1 < _NCH)
                def _():
                    issue(ch + 1, 1 - b)
                wait_b(b)

                def vloop(t, carry):
                    rv = rbuf[pl.ds(b * _CH + t * 16, 16)]
                    cv = cbuf[pl.ds(b * _CH + t * 16, 16)]
                    if part == 0:
                        process_vreg(rv, cv)
                    else:
                        process_vreg(cv, rv)
                    return carry
                lax.fori_loop(0, _CH // 16, vloop, 0)

            def chunk_pair(p, carry):
                one_chunk(p * 2, 0)
                one_chunk(p * 2 + 1, 1)
                return carry
            lax.fori_loop(0, _NCH // 2, chunk_pair, 0)

        run_part(0)
        run_part(1)

        def loop_body(g, carry):
            lc = iota + g * 16
            cur = cntw[pl.ds(g * 16, 16)]
            keep = cur < nbv
            rankc = jnp.minimum(cur, nbpm1)
            plsc.store_scatter(tblw, [lc * _NBP + rankc], basev + lc,
                               mask=keep)
            cntw[pl.ds(g * 16, 16)] = jnp.minimum(cur + 1, nbv)
            return carry
        lax.fori_loop(0, _PW // 16, loop_body, 0)

        pltpu.sync_copy(tblw, tbl_hbm.at[pl.ds(base * _NBP, _PW * _NBP)])
        pltpu.sync_copy(cntw, cnt_hbm.at[pl.ds(base, _PW)])

    tbl_flat, cnt = body(edge_index[0], edge_index[1])
    return tbl_flat.reshape(_NP, _NBP), cnt


def _sc_attention(q, k, v, tbl, cnt):
    mesh = plsc.VectorSubcoreMesh(core_axis_name="c", subcore_axis_name="s")

    @functools.partial(
        pl.kernel,
        mesh=mesh,
        out_type=jax.ShapeDtypeStruct((_NP, _D), jnp.float32),
        compiler_params=pltpu.CompilerParams(needs_layout_passes=False),
        scratch_types=[
            pltpu.VMEM((_PW + 16,), jnp.int32),
            pltpu.VMEM((2, _NBP), jnp.int32),
            pltpu.VMEM((2, _NBP, _D), jnp.float32),
            pltpu.VMEM((2, _NBP, _D), jnp.float32),
            pltpu.VMEM((2, _D), jnp.float32),
            pltpu.VMEM((_NH * 64 + 16,), jnp.float32),
            pltpu.VMEM((2, _D), jnp.float32),
            pltpu.SemaphoreType.DMA,
            pltpu.SemaphoreType.DMA,
            pltpu.SemaphoreType.DMA,
            pltpu.SemaphoreType.DMA,
            pltpu.SemaphoreType.DMA,
            pltpu.SemaphoreType.DMA,
            pltpu.SemaphoreType.DMA,
            pltpu.SemaphoreType.DMA,
            pltpu.SemaphoreType.DMA,
            pltpu.SemaphoreType.DMA,
        ],
    )
    def body(q_hbm, k_hbm, v_hbm, tbl_hbm, cnt_hbm, out_hbm,
             cnt_v, idx2, kbuf, vbuf, qbuf, sbuf, obuf,
             semt0, semt1, semq0, semq1, semk0, semk1,
             semv0, semv1, semo0, semo1):
        semt = (semt0, semt1)
        semq = (semq0, semq1)
        semk = (semk0, semk1)
        semv = (semv0, semv1)
        semo = (semo0, semo1)
        wid = lax.axis_index("s") * 2 + lax.axis_index("c")
        base = wid * _PW
        pltpu.sync_copy(cnt_hbm.at[pl.ds(base, _PW)],
                        cnt_v.at[pl.ds(0, _PW)])
        iota = lax.iota(jnp.int32, 16)

        def issue_a(n, b):
            u = base + n
            pltpu.async_copy(tbl_hbm.at[u], idx2.at[b], semt[b])
            pltpu.async_copy(q_hbm.at[u], qbuf.at[b], semq[b])

        def wait_a(b):
            pltpu.make_async_copy(tbl_hbm.at[0], idx2.at[b],
                                  semt[b]).wait()
            pltpu.make_async_copy(q_hbm.at[0], qbuf.at[b], semq[b]).wait()

        def issue_kv(b):
            pltpu.async_copy(k_hbm.at[idx2.at[b]], kbuf.at[b], semk[b])
            pltpu.async_copy(v_hbm.at[idx2.at[b]], vbuf.at[b], semv[b])

        def wait_kv(b):
            pltpu.make_async_copy(k_hbm.at[pl.ds(0, _NBP)], kbuf.at[b],
                                  semk[b]).wait()
            pltpu.make_async_copy(v_hbm.at[pl.ds(0, _NBP)], vbuf.at[b],
                                  semv[b]).wait()

        def wait_o(b):
            pltpu.make_async_copy(obuf.at[b], out_hbm.at[0],
                                  semo[b]).wait()

        issue_a(0, 0)
        wait_a(0)
        issue_kv(0)
        issue_a(1, 1)

        def one_node(i, b):
            u = base + i
            nb = 1 - b

            @pl.when(i + 1 < _PW)
            def _():
                wait_a(nb)
                issue_kv(nb)

            wait_kv(b)
            cntv = jnp.full((16,), cnt_v[pl.ds(i, 16)][0], dtype=jnp.int32)

            def score_body(he, c2):
                h = he // 4
                e = he % 4
                qv = qbuf[b, pl.ds(h * _HD, _HD)]
                rowv = iota + e * 16
                rowc = jnp.minimum(rowv, _NBP - 1)
                accs = [jnp.zeros((16,), jnp.float32) for _ in range(4)]
                for d in range(_HD):
                    col = jnp.full((16,), h * _HD + d, dtype=jnp.int32)
                    kcol = plsc.load_gather(kbuf.at[b], [rowc, col])
                    accs[d % 4] = accs[d % 4] + kcol * qv[d]
                sc = (accs[0] + accs[1]) + (accs[2] + accs[3])
                sv = jnp.where(rowv < cntv, sc, jnp.full((16,), -1e30))
                sbuf[pl.ds(he * 16, 16)] = sv
                return c2
            lax.fori_loop(0, _NH * 4, score_body, 0)

            for h in range(_NH):
                svs = [sbuf[pl.ds(h * 64 + 16 * e, 16)] for e in range(4)]
                m = jnp.max(jnp.maximum(jnp.maximum(svs[0], svs[1]),
                                        jnp.maximum(svs[2], svs[3])))
                exs = [jnp.exp(sv - m) for sv in svs]
                tot = jnp.sum(exs[0] + exs[1] + exs[2] + exs[3])
                for e in range(4):
                    sbuf[pl.ds(h * 64 + 16 * e, 16)] = exs[e] / tot

            aidx_base = (iota & 7) * 64

            def v_body(j, accs):
                av = plsc.load_gather(sbuf, [aidx_base + j])
                out = []
                for h in range(_NH):
                    vv = vbuf[b, j, pl.ds(h * _HD, _HD)]
                    out.append(accs[h] + vv * av[h])
                return tuple(out)
            accs = lax.fori_loop(
                0, _NB, v_body,
                tuple(jnp.zeros((16,), jnp.float32) for _ in range(_NH)))

            @pl.when(i >= 2)
            def _():
                wait_o(b)

            for h in range(_NH):
                obuf[b, pl.ds(h * _HD, _HD)] = accs[h]
            pltpu.async_copy(obuf.at[b], out_hbm.at[u], semo[b])

            @pl.when(i + 2 < _PW)
            def _():
                issue_a(i + 2, b)

        def pair_body(p, carry):
            one_node(p * 2, 0)
            one_node(p * 2 + 1, 1)
            return carry

        lax.fori_loop(0, _PW // 2, pair_body, 0)
        wait_o(0)
        wait_o(1)

    return body(q, k, v, tbl, cnt)


def kernel(query, key_value, edge_index, node_degrees, clustering_coeffs,
           pos_embedding, Wc, bc, Wd, bd, Wq, bq, Wk, bk, Wv, bv, Wo, bo,
           ln_q_g, ln_q_b, ln_kv_g, ln_kv_b, ln_out_g, ln_out_b):
    pad = _NP - _N
    qp = jnp.pad(query, ((0, pad), (0, 0)))
    kvp = jnp.pad(key_value, ((0, pad), (0, 0)))
    pep = jnp.pad(pos_embedding, ((0, pad), (0, 0)))
    degc = jnp.pad(node_degrees, (0, pad))[:, None]
    clusc = jnp.pad(clustering_coeffs, (0, pad))[:, None]

    tbl, cnt = _sc_table(edge_index)

    qpe, q, k, v = _dense_pre(
        qp, kvp, degc, clusc, pep,
        Wc.reshape(1, _D // 4), bc.reshape(1, _D // 4),
        Wd.reshape(1, _D // 4), bd.reshape(1, _D // 4),
        Wq, bq.reshape(1, _D), Wk, bk.reshape(1, _D),
        Wv, bv.reshape(1, _D),
        ln_q_g.reshape(1, _D), ln_q_b.reshape(1, _D),
        ln_kv_g.reshape(1, _D), ln_kv_b.reshape(1, _D))

    attn = _sc_attention(q, k, v, tbl, cnt)

    out = _dense_post(attn, qpe, Wo, bo.reshape(1, _D),
                      ln_out_g.reshape(1, _D), ln_out_b.reshape(1, _D))
    return out[:_N]


# speedup vs baseline: 82.7668x; 1.4596x over previous
import functools

import numpy as np
import jax
import jax.numpy as jnp
from jax import lax
from jax.experimental import pallas as pl
from jax.experimental.pallas import tpu as pltpu
from jax.experimental.pallas import tpu_sc as plsc

_N = 10000
_D = 128
_NH = 8
_HD = 16
_NB = 50
_SCALE = float(1.0 / np.sqrt(_HD))
_NW = 32
_PW = 320
_NP = _NW * _PW
_NBP = 56
_BS = 512
_EPS = 1e-5


def _ln_blk(x, g, b):
    m = jnp.mean(x, axis=-1, keepdims=True)
    v = jnp.mean((x - m) ** 2, axis=-1, keepdims=True)
    return (x - m) * jax.lax.rsqrt(v + _EPS) * g + b


def _dense_pre_body(q_ref, kv_ref, deg_ref, clus_ref, pe_ref,
                    wc_ref, bc_ref, wd_ref, bd_ref,
                    wq_ref, bq_ref, wk_ref, bk_ref, wv_ref, bv_ref,
                    g1_ref, b1_ref, g2_ref, b2_ref,
                    qpe_ref, qo_ref, ko_ref, vo_ref):
    pe = pe_ref[...]
    ce = clus_ref[...] * wc_ref[...] + bc_ref[...]
    de = deg_ref[...] * wd_ref[...] + bd_ref[...]
    penc = jnp.concatenate([pe[:, : _D // 2], ce, de], axis=-1)
    qpe = q_ref[...] + penc
    kvpe = kv_ref[...] + penc
    qn = _ln_blk(qpe, g1_ref[...], b1_ref[...])
    kvn = _ln_blk(kvpe, g2_ref[...], b2_ref[...])
    dn = (((1,), (1,)), ((), ()))
    qo_ref[...] = (lax.dot_general(qn, wq_ref[...], dn,
                                   preferred_element_type=jnp.float32)
                   + bq_ref[...]) * _SCALE
    ko_ref[...] = lax.dot_general(kvn, wk_ref[...], dn,
                                  preferred_element_type=jnp.float32) + bk_ref[...]
    vo_ref[...] = lax.dot_general(kvn, wv_ref[...], dn,
                                  preferred_element_type=jnp.float32) + bv_ref[...]
    qpe_ref[...] = qpe


def _dense_pre(qp, kvp, degc, clusc, pep, wct, bct, wdt, bdt,
               wq, bq2, wk, bk2, wv, bv2, g1, b1, g2, b2, interpret=False):
    nblk = _NP // _BS
    row_spec = pl.BlockSpec((_BS, _D), lambda i: (i, 0))
    col_spec = pl.BlockSpec((_BS, 1), lambda i: (i, 0))

    def full(shape):
        return pl.BlockSpec(shape, lambda i: (0,) * len(shape))

    return pl.pallas_call(
        _dense_pre_body,
        grid=(nblk,),
        in_specs=[row_spec, row_spec, col_spec, col_spec, row_spec,
                  full((1, _D // 4)), full((1, _D // 4)),
                  full((1, _D // 4)), full((1, _D // 4)),
                  full((_D, _D)), full((1, _D)),
                  full((_D, _D)), full((1, _D)),
                  full((_D, _D)), full((1, _D)),
                  full((1, _D)), full((1, _D)), full((1, _D)), full((1, _D))],
        out_specs=[row_spec, row_spec, row_spec, row_spec],
        out_shape=[jax.ShapeDtypeStruct((_NP, _D), jnp.float32)] * 4,
        interpret=interpret,
    )(qp, kvp, degc, clusc, pep, wct, bct, wdt, bdt,
      wq, bq2, wk, bk2, wv, bv2, g1, b1, g2, b2)


def _dense_post_body(attn_ref, qpe_ref, wo_ref, bo_ref, g_ref, b_ref, o_ref):
    dn = (((1,), (1,)), ((), ()))
    out = lax.dot_general(attn_ref[...], wo_ref[...], dn,
                          preferred_element_type=jnp.float32) + bo_ref[...]
    o_ref[...] = _ln_blk(qpe_ref[...] + out, g_ref[...], b_ref[...])


def _dense_post(attn, qpe, wo, bo2, g, b, interpret=False):
    nblk = _NP // _BS
    row_spec = pl.BlockSpec((_BS, _D), lambda i: (i, 0))

    def full(shape):
        return pl.BlockSpec(shape, lambda i: (0,) * len(shape))

    return pl.pallas_call(
        _dense_post_body,
        grid=(nblk,),
        in_specs=[row_spec, row_spec, full((_D, _D)), full((1, _D)),
                  full((1, _D)), full((1, _D))],
        out_specs=row_spec,
        out_shape=jax.ShapeDtypeStruct((_NP, _D), jnp.float32),
        interpret=interpret,
    )(attn, qpe, wo, bo2, g, b)


_CH = 2000
_NCH = 320000 // _CH


def _sc_table(edge_index):
    mesh = plsc.VectorSubcoreMesh(core_axis_name="c", subcore_axis_name="s")

    @functools.partial(
        pl.kernel,
        mesh=mesh,
        out_type=(jax.ShapeDtypeStruct((_NP * _NBP,), jnp.int32),
                  jax.ShapeDtypeStruct((_NP,), jnp.int32)),
        compiler_params=pltpu.CompilerParams(needs_layout_passes=False),
        scratch_types=[
            pltpu.VMEM((_PW * _NBP,), jnp.int32),
            pltpu.VMEM((_PW,), jnp.int32),
            pltpu.VMEM((2 * _CH,), jnp.int32),
            pltpu.VMEM((2 * _CH,), jnp.int32),
            pltpu.SemaphoreType.DMA,
            pltpu.SemaphoreType.DMA,
            pltpu.SemaphoreType.DMA,
            pltpu.SemaphoreType.DMA,
        ],
    )
    def body(r_hbm, c_hbm, tbl_hbm, cnt_hbm, tblw, cntw, rbuf, cbuf,
             semr0, semr1, semc0, semc1):
        semr = (semr0, semr1)
        semc = (semc0, semc1)
        wid = lax.axis_index("s") * 2 + lax.axis_index("c")
        base = wid * _PW
        iota = lax.iota(jnp.int32, 16)
        basev = jnp.full((16,), base, jnp.int32)

        def init_body(n, carry):
            nv = basev + jnp.full((16,), n, jnp.int32)
            tblw[pl.ds(n * _NBP, 16)] = nv
            tblw[pl.ds(n * _NBP + 16, 16)] = nv
            tblw[pl.ds(n * _NBP + 32, 16)] = nv
            tblw[pl.ds(n * _NBP + 40, 16)] = nv
            return carry
        lax.fori_loop(0, _PW, init_body, 0)

        def czero(g, carry):
            cntw[pl.ds(g * 16, 16)] = jnp.zeros((16,), jnp.int32)
            return carry
        lax.fori_loop(0, _PW // 16, czero, 0)

        def issue(ch, b):
            pltpu.async_copy(r_hbm.at[pl.ds(ch * _CH, _CH)],
                             rbuf.at[pl.ds(b * _CH, _CH)], semr[b])
            pltpu.async_copy(c_hbm.at[pl.ds(ch * _CH, _CH)],
                             cbuf.at[pl.ds(b * _CH, _CH)], semc[b])

        def wait_b(b):
            pltpu.make_async_copy(r_hbm.at[pl.ds(0, _CH)],
                                  rbuf.at[pl.ds(b * _CH, _CH)],
                                  semr[b]).wait()
            pltpu.make_async_copy(c_hbm.at[pl.ds(0, _CH)],
                                  cbuf.at[pl.ds(b * _CH, _CH)],
                                  semc[b]).wait()

        nbv = jnp.full((16,), _NB, jnp.int32)
        pwv = jnp.full((16,), _PW, jnp.int32)
        nbpm1 = jnp.full((16,), _NBP - 1, jnp.int32)
        zv = jnp.zeros((16,), jnp.int32)

        def process_vreg(src, tgt):
            local = src - basev
            m = (local >= zv) & (local < pwv)
            lc = jnp.minimum(jnp.maximum(local, zv), pwv - 1)
            cur = plsc.load_gather(cntw, [lc])
            occ, lastm = plsc.scan_count(lc, mask=m)
            rank = cur + occ - 1
            keep = m & (rank < nbv)
            rankc = jnp.minimum(rank, nbpm1)
            plsc.store_scatter(tblw, [lc * _NBP + rankc], tgt, mask=keep)
            plsc.store_scatter(cntw, [lc], cur + occ, mask=m & lastm)

        def run_part(part):
            issue(0, 0)

            def one_chunk(ch, b):
                @pl.when(ch + 1 < _NCH)
                def _():
                    issue(ch + 1, 1 - b)
                wait_b(b)

                def vloop(t, carry):
                    rv = rbuf[pl.ds(b * _CH + t * 16, 16)]
                    cv = cbuf[pl.ds(b * _CH + t * 16, 16)]
                    if part == 0:
                        process_vreg(rv, cv)
                    else:
                        process_vreg(cv, rv)
                    return carry
                lax.fori_loop(0, _CH // 16, vloop, 0)

            def chunk_pair(p, carry):
                one_chunk(p * 2, 0)
                one_chunk(p * 2 + 1, 1)
                return carry
            lax.fori_loop(0, _NCH // 2, chunk_pair, 0)

        run_part(0)
        run_part(1)

        def loop_body(g, carry):
            lc = iota + g * 16
            cur = cntw[pl.ds(g * 16, 16)]
            keep = cur < nbv
            rankc = jnp.minimum(cur, nbpm1)
            plsc.store_scatter(tblw, [lc * _NBP + rankc], basev + lc,
                               mask=keep)
            cntw[pl.ds(g * 16, 16)] = jnp.minimum(cur + 1, nbv)
            return carry
        lax.fori_loop(0, _PW // 16, loop_body, 0)

        pltpu.sync_copy(tblw, tbl_hbm.at[pl.ds(base * _NBP, _PW * _NBP)])
        pltpu.sync_copy(cntw, cnt_hbm.at[pl.ds(base, _PW)])

    tbl_flat, cnt = body(edge_index[0], edge_index[1])
    return tbl_flat.reshape(_NP, _NBP), cnt


def _sc_attention(q, k, v, tbl, cnt):
    mesh = plsc.VectorSubcoreMesh(core_axis_name="c", subcore_axis_name="s")

    @functools.partial(
        pl.kernel,
        mesh=mesh,
        out_type=jax.ShapeDtypeStruct((_NP, _D), jnp.float32),
        compiler_params=pltpu.CompilerParams(needs_layout_passes=False),
        scratch_types=[
            pltpu.VMEM((_PW + 16,), jnp.int32),
            pltpu.VMEM((2, _NBP), jnp.int32),
            pltpu.VMEM((2, _NBP, _D), jnp.float32),
            pltpu.VMEM((2, _NBP, _D), jnp.float32),
            pltpu.VMEM((2, _D), jnp.float32),
            pltpu.VMEM((_NH * 65 + 16,), jnp.float32),
            pltpu.VMEM((2, _D), jnp.float32),
            pltpu.SemaphoreType.DMA,
            pltpu.SemaphoreType.DMA,
            pltpu.SemaphoreType.DMA,
            pltpu.SemaphoreType.DMA,
            pltpu.SemaphoreType.DMA,
            pltpu.SemaphoreType.DMA,
            pltpu.SemaphoreType.DMA,
            pltpu.SemaphoreType.DMA,
            pltpu.SemaphoreType.DMA,
            pltpu.SemaphoreType.DMA,
        ],
    )
    def body(q_hbm, k_hbm, v_hbm, tbl_hbm, cnt_hbm, out_hbm,
             cnt_v, idx2, kbuf, vbuf, qbuf, sbuf, obuf,
             semt0, semt1, semq0, semq1, semk0, semk1,
             semv0, semv1, semo0, semo1):
        semt = (semt0, semt1)
        semq = (semq0, semq1)
        semk = (semk0, semk1)
        semv = (semv0, semv1)
        semo = (semo0, semo1)
        wid = lax.axis_index("s") * 2 + lax.axis_index("c")
        base = wid * _PW
        pltpu.sync_copy(cnt_hbm.at[pl.ds(base, _PW)],
                        cnt_v.at[pl.ds(0, _PW)])
        iota = lax.iota(jnp.int32, 16)

        def issue_a(n, b):
            u = base + n
            pltpu.async_copy(tbl_hbm.at[u], idx2.at[b], semt[b])
            pltpu.async_copy(q_hbm.at[u], qbuf.at[b], semq[b])

        def wait_a(b):
            pltpu.make_async_copy(tbl_hbm.at[0], idx2.at[b],
                                  semt[b]).wait()
            pltpu.make_async_copy(q_hbm.at[0], qbuf.at[b], semq[b]).wait()

        def issue_kv(b):
            pltpu.async_copy(k_hbm.at[idx2.at[b]], kbuf.at[b], semk[b])
            pltpu.async_copy(v_hbm.at[idx2.at[b]], vbuf.at[b], semv[b])

        def wait_kv(b):
            pltpu.make_async_copy(k_hbm.at[pl.ds(0, _NBP)], kbuf.at[b],
                                  semk[b]).wait()
            pltpu.make_async_copy(v_hbm.at[pl.ds(0, _NBP)], vbuf.at[b],
                                  semv[b]).wait()

        def wait_o(b):
            pltpu.make_async_copy(obuf.at[b], out_hbm.at[0],
                                  semo[b]).wait()

        issue_a(0, 0)
        wait_a(0)
        issue_kv(0)
        issue_a(1, 1)

        def one_node(i, b):
            u = base + i
            nb = 1 - b

            @pl.when(i + 1 < _PW)
            def _():
                wait_a(nb)
                issue_kv(nb)

            wait_kv(b)
            cntv = jnp.full((16,), cnt_v[pl.ds(i, 16)][0], dtype=jnp.int32)
            rotidx = [jnp.bitwise_and(iota + t, 15) for t in range(_HD)]
            rowvs = [iota + e * 16 for e in range(4)]
            rowcs = [jnp.minimum(rv, _NBP - 1) for rv in rowvs]

            def score_h(h, c2):
                h16 = jnp.full((16,), h * _HD, dtype=jnp.int32)
                accs = [jnp.zeros((16,), jnp.float32) for _ in range(4)]
                for t in range(_HD):
                    colv = h16 + rotidx[t]
                    qg = plsc.load_gather(qbuf.at[b], [colv])
                    for e in range(4):
                        kg = plsc.load_gather(kbuf.at[b], [rowcs[e], colv])
                        accs[e] = accs[e] + kg * qg
                for e in range(4):
                    sv = jnp.where(rowvs[e] < cntv, accs[e],
                                   jnp.full((16,), -1e30))
                    sbuf[pl.ds(h * 65 + e * 16, 16)] = sv
                return c2
            lax.fori_loop(0, _NH, score_h, 0)

            for h in range(_NH):
                svs = [sbuf[pl.ds(h * 65 + 16 * e, 16)] for e in range(4)]
                m = jnp.max(jnp.maximum(jnp.maximum(svs[0], svs[1]),
                                        jnp.maximum(svs[2], svs[3])))
                exs = [jnp.exp(sv - m) for sv in svs]
                tot = jnp.sum(exs[0] + exs[1] + exs[2] + exs[3])
                for e in range(4):
                    sbuf[pl.ds(h * 65 + 16 * e, 16)] = exs[e] / tot

            aidx_base = (iota & 7) * 65

            def v_body(j, accs):
                av = plsc.load_gather(sbuf, [aidx_base + j])
                out = []
                for h in range(_NH):
                    vv = vbuf[b, j, pl.ds(h * _HD, _HD)]
                    out.append(accs[h] + vv * av[h])
                return tuple(out)
            accs = lax.fori_loop(
                0, _NB, v_body,
                tuple(jnp.zeros((16,), jnp.float32) for _ in range(_NH)))

            @pl.when(i >= 2)
            def _():
                wait_o(b)

            for h in range(_NH):
                obuf[b, pl.ds(h * _HD, _HD)] = accs[h]
            pltpu.async_copy(obuf.at[b], out_hbm.at[u], semo[b])

            @pl.when(i + 2 < _PW)
            def _():
                issue_a(i + 2, b)

        def pair_body(p, carry):
            one_node(p * 2, 0)
            one_node(p * 2 + 1, 1)
            return carry

        lax.fori_loop(0, _PW // 2, pair_body, 0)
        wait_o(0)
        wait_o(1)

    return body(q, k, v, tbl, cnt)


def kernel(query, key_value, edge_index, node_degrees, clustering_coeffs,
           pos_embedding, Wc, bc, Wd, bd, Wq, bq, Wk, bk, Wv, bv, Wo, bo,
           ln_q_g, ln_q_b, ln_kv_g, ln_kv_b, ln_out_g, ln_out_b):
    pad = _NP - _N
    qp = jnp.pad(query, ((0, pad), (0, 0)))
    kvp = jnp.pad(key_value, ((0, pad), (0, 0)))
    pep = jnp.pad(pos_embedding, ((0, pad), (0, 0)))
    degc = jnp.pad(node_degrees, (0, pad))[:, None]
    clusc = jnp.pad(clustering_coeffs, (0, pad))[:, None]

    tbl, cnt = _sc_table(edge_index)

    qpe, q, k, v = _dense_pre(
        qp, kvp, degc, clusc, pep,
        Wc.reshape(1, _D // 4), bc.reshape(1, _D // 4),
        Wd.reshape(1, _D // 4), bd.reshape(1, _D // 4),
        Wq, bq.reshape(1, _D), Wk, bk.reshape(1, _D),
        Wv, bv.reshape(1, _D),
        ln_q_g.reshape(1, _D), ln_q_b.reshape(1, _D),
        ln_kv_g.reshape(1, _D), ln_kv_b.reshape(1, _D))

    attn = _sc_attention(q, k, v, tbl, cnt)

    out = _dense_post(attn, qpe, Wo, bo.reshape(1, _D),
                      ln_out_g.reshape(1, _D), ln_out_b.reshape(1, _D))
    return out[:_N]
